# Initial kernel scaffold; baseline (speedup 1.0000x reference)
import jax
import jax.numpy as jnp
from jax.experimental import pallas as pl


def kernel(input_ids, attention_mask, tok_embs, tok_embs_weights):
    raise NotImplementedError("write your pallas kernel here")


# speedup vs baseline: 2.2136x; 2.2136x over previous
import functools

import jax
import jax.numpy as jnp
from jax import lax
from jax.experimental import pallas as pl
from jax.experimental.pallas import tpu as pltpu
from jax.experimental.pallas import tpu_sc as plsc


def _build_sc_kernel(B, L, LP, D, rows_per_w, nc):
    mesh = plsc.VectorSubcoreMesh(core_axis_name="c", subcore_axis_name="s")
    n_lchunk = LP // 16
    n_dchunk = D // 16
    LG = ((L + 7) // 8) * 8

    @functools.partial(
        pl.kernel,
        mesh=mesh,
        out_type=jax.ShapeDtypeStruct((B, D), jnp.float32),
        compiler_params=pltpu.CompilerParams(needs_layout_passes=False),
        scratch_types=[
            pltpu.VMEM((2, LP), jnp.int32),
            pltpu.VMEM((2, LP), jnp.float32),
            pltpu.VMEM((2, LP), jnp.int32),
            pltpu.VMEM((2, LG, D), jnp.float32),
            pltpu.VMEM((2, D), jnp.float32),
            pltpu.VMEM((16,), jnp.float32),
            pltpu.SemaphoreType.DMA,
        ],
    )
    def k(table, ids, mask, wvec, out, ids_v, w_v, mask_v, rows_v, out_v, red_v, sem):
        wid = lax.axis_index("s") * nc + lax.axis_index("c")
        base = wid * rows_per_w

        zf16 = jnp.zeros((16,), jnp.float32)
        for t in range(2):
            for c in range(n_lchunk):
                w_v[t, pl.ds(c * 16, 16)] = zf16

        def row_body(i, carry):
            b = base + i
            t = 0
            pltpu.sync_copy(ids.at[b], ids_v.at[t])
            pltpu.sync_copy(mask.at[b], mask_v.at[t])
            idx = ids_v.at[t, pl.ds(0, LG)]
            pltpu.async_copy(wvec.at[idx], w_v.at[t, pl.ds(0, LG)], sem).wait()
            pltpu.async_copy(table.at[idx], rows_v.at[t], sem).wait()

            total = jnp.zeros((16,), jnp.float32)
            for c in range(n_lchunk):
                sl = pl.ds(c * 16, 16)
                wm = w_v[t, sl] * mask_v[t, sl].astype(jnp.float32)
                w_v[t, sl] = wm
                total = total + wm
            lanes = lax.iota(jnp.int32, 16)
            for s in (8, 4, 2, 1):
                red_v[...] = total
                total = total + plsc.load_gather(red_v, [lanes ^ s])
            inv = jnp.float32(1.0) / total

            for q in range(4):
                nch = n_dchunk // 4
                qoff = q * nch * 16

                def acc_body(l, accs):
                    li = jnp.full((16,), l, jnp.int32)
                    ti = jnp.full((16,), t, jnp.int32)
                    wl = plsc.load_gather(w_v, [ti, li])
                    return tuple(
                        accs[j] + wl * rows_v[t, l, pl.ds(qoff + j * 16, 16)]
                        for j in range(nch)
                    )

                accs = lax.fori_loop(0, L, acc_body, tuple(zf16 for _ in range(nch)))
                for j in range(nch):
                    out_v[t, pl.ds(qoff + j * 16, 16)] = accs[j] * inv

            pltpu.sync_copy(out_v.at[t], out.at[b])
            return carry

        lax.fori_loop(0, rows_per_w, row_body, 0)

    return k


def kernel(input_ids, attention_mask, tok_embs, tok_embs_weights):
    B, L = input_ids.shape
    V, D = tok_embs.shape
    info = plsc.get_sparse_core_info()
    nw = info.num_cores * info.num_subcores
    assert B % nw == 0 and D % 64 == 0
    LP = ((L + 15) // 16) * 16
    ids_p = jnp.pad(input_ids.astype(jnp.int32), ((0, 0), (0, LP - L)))
    mask_p = jnp.pad(attention_mask.astype(jnp.int32), ((0, 0), (0, LP - L)))
    k = _build_sc_kernel(B, L, LP, D, B // nw, info.num_cores)
    return k(tok_embs, ids_p, mask_p, tok_embs_weights)
